# baseline (speedup 1.0000x reference)
import functools

import jax
import jax.numpy as jnp
from jax import lax
from jax.experimental import pallas as pl
from jax.experimental.pallas import tpu as pltpu
from jax.experimental.pallas import tpu_sc as plsc

B = 4
V = 10000
E = 320000
F = 128
FQ = 32
NQ = F // FQ
NS = 16
G = 128
EPT = E // NS
NCHUNK = 162
EPT_PAD = NCHUNK * G
NITER = NCHUNK // 6
RPTX = 632
RPTL = V - 15 * RPTX


def _sc_spmm_body(xq_hbm, cols_hbm, rows_hbm, valsbc_hbm, zeros_hbm,
                  s1_hbm, s2_hbm,
                  buf0, buf1, cols_v, rows_v, idxb, gbuf, vbuf,
                  *sems):
    c = lax.axis_index("c")
    s = lax.axis_index("s")
    gsems = sems[:6]
    ssems = sems[6:]

    pltpu.sync_copy(cols_hbm.at[s], cols_v)
    pltpu.sync_copy(rows_hbm.at[s], rows_v)

    def vals_rowbase(j):
        return (s * NCHUNK + j) * G

    def compute_idx(slot, j, b_off):
        for t in range(G // 16):
            idxb[slot, pl.ds(16 * t, 16)] = cols_v[j, pl.ds(16 * t, 16)] + b_off

    def issue_gather(slot, j, srcb, b_off):
        compute_idx(slot, j, b_off)
        pltpu.async_copy(srcb.at[idxb.at[slot]], gbuf.at[slot], gsems[slot])
        pltpu.async_copy(valsbc_hbm.at[pl.ds(vals_rowbase(j), G)],
                         vbuf.at[slot], gsems[slot])

    def wait_gather(slot, j, srcb):
        pltpu.make_async_copy(srcb.at[idxb.at[slot]], gbuf.at[slot],
                              gsems[slot]).wait()
        pltpu.make_async_copy(valsbc_hbm.at[pl.ds(vals_rowbase(j), G)],
                              vbuf.at[slot], gsems[slot]).wait()

    def issue_scatter(slot, j, accb):
        pltpu.async_copy(gbuf.at[slot], accb.at[rows_v.at[j]], ssems[slot],
                         add=True)

    def wait_scatter(slot, j, accb):
        pltpu.make_async_copy(gbuf.at[slot], accb.at[rows_v.at[j]],
                              ssems[slot]).wait()

    def scale_grp(slot_base):
        def body(qq, carry):
            slot = slot_base + qq

            @plsc.parallel_loop(0, G, 1, unroll=4)
            def _pe(e):
                vv = vbuf[slot, e]
                for jj in range(FQ // 16):
                    cur = gbuf[slot, e, pl.ds(16 * jj, 16)]
                    gbuf[slot, e, pl.ds(16 * jj, 16)] = cur * vv

            return carry

        lax.fori_loop(0, 3, body, 0)

    def edge_sweep(srcb, accb, b_off):
        for q in range(3):
            issue_gather(q, q, srcb, b_off)
        for q in range(3):
            issue_gather(3 + q, 3 + q, srcb, b_off)

        def body(m, carry):
            cb = 6 * m
            for q in range(3):
                wait_gather(q, cb + q, srcb)
            scale_grp(0)
            for q in range(3):
                issue_scatter(q, cb + q, accb)
            for q in range(3):
                wait_gather(3 + q, cb + 3 + q, srcb)
            scale_grp(3)
            for q in range(3):
                issue_scatter(3 + q, cb + 3 + q, accb)

            @pl.when(m < NITER - 1)
            def _refill():
                for q in range(3):
                    wait_scatter(q, cb + q, accb)
                    issue_gather(q, cb + 6 + q, srcb, b_off)
                for q in range(3):
                    wait_scatter(3 + q, cb + 3 + q, accb)
                    issue_gather(3 + q, cb + 9 + q, srcb, b_off)

            return carry

        lax.fori_loop(0, NITER, body, 0)
        for q in range(3):
            wait_scatter(q, NCHUNK - 6 + q, accb)
        for q in range(3):
            wait_scatter(3 + q, NCHUNK - 3 + q, accb)

    def zero_buf(buf):
        @pl.when(s < NS - 1)
        def _z():
            pltpu.sync_copy(zeros_hbm, buf.at[pl.ds(s * RPTX, RPTX)])

        @pl.when(s == NS - 1)
        def _zl():
            pltpu.sync_copy(zeros_hbm.at[pl.ds(0, RPTL)],
                            buf.at[pl.ds(s * RPTX, RPTL)])

    def stage_src(qi, b_off, buf):
        @pl.when(s < NS - 1)
        def _s():
            pltpu.sync_copy(xq_hbm.at[qi, pl.ds(b_off + s * RPTX, RPTX)],
                            buf.at[pl.ds(s * RPTX, RPTX)])

        @pl.when(s == NS - 1)
        def _sl():
            pltpu.sync_copy(xq_hbm.at[qi, pl.ds(b_off + s * RPTX, RPTL)],
                            buf.at[pl.ds(s * RPTX, RPTL)])

    def publish(buf, dst_hbm, qi, b_off):
        @pl.when(s < NS - 1)
        def _p():
            pltpu.sync_copy(buf.at[pl.ds(s * RPTX, RPTX)],
                            dst_hbm.at[qi, pl.ds(b_off + s * RPTX, RPTX)])

        @pl.when(s == NS - 1)
        def _pl():
            pltpu.sync_copy(buf.at[pl.ds(s * RPTX, RPTL)],
                            dst_hbm.at[qi, pl.ds(b_off + s * RPTX, RPTL)])

    def pass_body(qi, carry_b_off):
        b_off = carry_b_off
        zero_buf(buf1)
        plsc.subcore_barrier()
        edge_sweep(xq_hbm.at[qi], buf1, b_off)
        plsc.subcore_barrier()
        publish(buf1, s1_hbm, qi, b_off)
        zero_buf(buf0)
        plsc.subcore_barrier()
        edge_sweep(s1_hbm.at[qi], buf0, b_off)
        plsc.subcore_barrier()
        publish(buf0, s2_hbm, qi, b_off)
        plsc.subcore_barrier()
        return carry_b_off

    def batch_body(i_b, carry):
        b_off = (2 * c + i_b) * V
        lax.fori_loop(0, NQ, pass_body, b_off)
        return carry

    lax.fori_loop(0, 2, batch_body, 0)


_sc_spmm = functools.partial(
    pl.kernel,
    out_type=[jax.ShapeDtypeStruct((NQ, B * V, FQ), jnp.float32)] * 2,
    mesh=plsc.VectorSubcoreMesh(core_axis_name="c", subcore_axis_name="s"),
    compiler_params=pltpu.CompilerParams(use_tc_tiling_on_sc=False),
    scratch_types=[
        pltpu.VMEM_SHARED((V, FQ), jnp.float32),
        pltpu.VMEM_SHARED((V, FQ), jnp.float32),
        pltpu.VMEM((NCHUNK, G), jnp.int32),
        pltpu.VMEM((NCHUNK, G), jnp.int32),
        pltpu.VMEM((6, G), jnp.int32),
        pltpu.VMEM((6, G, FQ), jnp.float32),
        pltpu.VMEM((6, G, 16), jnp.float32),
    ] + [pltpu.SemaphoreType.DMA] * 12,
)(_sc_spmm_body)


def _mm_body(x0_ref, s1q0, s1q1, s1q2, s1q3, s2q0, s2q1, s2q2, s2q3,
             w_ref, b_ref, o_ref):
    acc = jnp.dot(x0_ref[...], w_ref[0], preferred_element_type=jnp.float32)
    for i, r in enumerate((s1q0, s1q1, s1q2, s1q3)):
        acc += jnp.dot(r[...], w_ref[1, FQ * i:FQ * (i + 1)],
                       preferred_element_type=jnp.float32)
    for i, r in enumerate((s2q0, s2q1, s2q2, s2q3)):
        acc += jnp.dot(r[...], w_ref[2, FQ * i:FQ * (i + 1)],
                       preferred_element_type=jnp.float32)
    o_ref[...] = acc + b_ref[...]


_MM_BLK = 800


def _tc_matmul(x0, s1qs, s2qs, wadj, bias2d):
    grid = (B * V) // _MM_BLK
    q_spec = pl.BlockSpec((_MM_BLK, FQ), lambda i: (i, 0))
    return pl.pallas_call(
        _mm_body,
        grid=(grid,),
        in_specs=[pl.BlockSpec((_MM_BLK, F), lambda i: (i, 0))]
                 + [q_spec] * 8
                 + [pl.BlockSpec((3, F, F), lambda i: (0, 0, 0)),
                    pl.BlockSpec((1, F), lambda i: (0, 0))],
        out_specs=pl.BlockSpec((_MM_BLK, F), lambda i: (i, 0)),
        out_shape=jax.ShapeDtypeStruct((B * V, F), jnp.float32),
    )(x0, *s1qs, *s2qs, wadj, bias2d)


def _pad_edges(arr, fill):
    slab = arr.reshape(NS, EPT)
    slab = jnp.pad(slab, ((0, 0), (0, EPT_PAD - EPT)), constant_values=fill)
    return slab.reshape(NS, NCHUNK, G)


def kernel(inputs, lap_rows, lap_cols, lap_vals, weight, bias):
    x_flat = inputs.reshape(B * V, F)
    x_q = jnp.stack([x_flat[:, FQ * i:FQ * (i + 1)] for i in range(NQ)], 0)
    cols_p = _pad_edges(lap_cols.astype(jnp.int32), 0)
    rows_p = _pad_edges(lap_rows.astype(jnp.int32), 0)
    vals_p = _pad_edges(lap_vals, 0.0)
    vals_bc = jnp.broadcast_to(vals_p.reshape(NS * NCHUNK * G, 1),
                               (NS * NCHUNK * G, 16))
    zeros = jnp.zeros((RPTX, FQ), jnp.float32)
    s1, s2 = _sc_spmm(x_q, cols_p, rows_p, vals_bc, zeros)
    s1qs = [s1[i] for i in range(NQ)]
    s2qs = [s2[i] for i in range(NQ)]
    wadj = jnp.stack([weight[:, 0, :] - weight[:, 2, :],
                      weight[:, 1, :],
                      2.0 * weight[:, 2, :]], axis=0)
    out_flat = _tc_matmul(x_flat, s1qs, s2qs, wadj, bias.reshape(1, F))
    return out_flat.reshape(B, V, F)


# speedup vs baseline: 1.1495x; 1.1495x over previous
import functools

import jax
import jax.numpy as jnp
from jax import lax
from jax.experimental import pallas as pl
from jax.experimental.pallas import tpu as pltpu
from jax.experimental.pallas import tpu_sc as plsc

B = 4
V = 10000
E = 320000
F = 128
FH = 64
NS = 16
G = 64
EPT = E // NS
NCHUNK = 324
EPT_PAD = NCHUNK * G
NITER = NCHUNK // 9
ACCV = 10240
RPT = ACCV // NS
RPT_LAST = V - 15 * RPT


def _sc_spmm_body(xlo_hbm, xhi_hbm, cols_hbm, rows_hbm, valsbc_hbm,
                  zeros_hbm, s1lo_hbm, s1hi_hbm, s2lo_hbm, s2hi_hbm,
                  acc, cols_v, rows_v, idxb, gbuf, vbuf,
                  *sems):
    c = lax.axis_index("c")
    s = lax.axis_index("s")
    gsems = sems[:9]
    ssems = sems[9:]

    pltpu.sync_copy(cols_hbm.at[s], cols_v)
    pltpu.sync_copy(rows_hbm.at[s], rows_v)

    def compute_idx(slot, j, b_off):
        for t in range(G // 16):
            idxb[slot, pl.ds(16 * t, 16)] = cols_v[j, pl.ds(16 * t, 16)] + b_off

    def vals_rowbase(j):
        return (s * NCHUNK + j) * G

    def issue_gather(slot, j, src, b_off):
        compute_idx(slot, j, b_off)
        pltpu.async_copy(src.at[idxb.at[slot]], gbuf.at[slot], gsems[slot])
        pltpu.async_copy(valsbc_hbm.at[pl.ds(vals_rowbase(j), G)],
                         vbuf.at[slot], gsems[slot])

    def wait_gather(slot, j, src):
        pltpu.make_async_copy(src.at[idxb.at[slot]], gbuf.at[slot],
                              gsems[slot]).wait()
        pltpu.make_async_copy(valsbc_hbm.at[pl.ds(vals_rowbase(j), G)],
                              vbuf.at[slot], gsems[slot]).wait()

    def issue_scatter(slot, j):
        pltpu.async_copy(gbuf.at[slot], acc.at[rows_v.at[j]], ssems[slot],
                         add=True)

    def wait_scatter(slot, j):
        pltpu.make_async_copy(gbuf.at[slot], acc.at[rows_v.at[j]],
                              ssems[slot]).wait()

    def scale_grp(slot_base):
        def body(qq, carry):
            slot = slot_base + qq

            @plsc.parallel_loop(0, G, 1, unroll=4)
            def _pe(e):
                vv = vbuf[slot, e]
                for jj in range(FH // 16):
                    cur = gbuf[slot, e, pl.ds(16 * jj, 16)]
                    gbuf[slot, e, pl.ds(16 * jj, 16)] = cur * vv

            return carry

        lax.fori_loop(0, 3, body, 0)

    def run_pass(src, dst, b):
        b_off = b * V
        pltpu.sync_copy(zeros_hbm, acc.at[pl.ds(s * RPT, RPT)])
        plsc.subcore_barrier()

        for q in range(3):
            issue_gather(q, q, src, b_off)
        for q in range(3):
            issue_gather(3 + q, 3 + q, src, b_off)

        def body(m, carry):
            cb = 9 * m
            for q in range(3):
                wait_gather(q, cb + q, src)
            scale_grp(0)
            for q in range(3):
                issue_scatter(q, cb + q)

            @pl.when(m >= 1)
            def _wc():
                for q in range(3):
                    wait_scatter(6 + q, cb - 3 + q)

            for q in range(3):
                issue_gather(6 + q, cb + 6 + q, src, b_off)

            for q in range(3):
                wait_gather(3 + q, cb + 3 + q, src)
            scale_grp(3)
            for q in range(3):
                issue_scatter(3 + q, cb + 3 + q)

            for q in range(3):
                wait_gather(6 + q, cb + 6 + q, src)
            scale_grp(6)
            for q in range(3):
                issue_scatter(6 + q, cb + 6 + q)

            @pl.when(m < NITER - 1)
            def _wab():
                for q in range(3):
                    wait_scatter(q, cb + q)
                    issue_gather(q, cb + 9 + q, src, b_off)
                for q in range(3):
                    wait_scatter(3 + q, cb + 3 + q)
                    issue_gather(3 + q, cb + 12 + q, src, b_off)

            return carry

        lax.fori_loop(0, NITER, body, 0)
        for q in range(3):
            wait_scatter(q, NCHUNK - 9 + q)
        for q in range(3):
            wait_scatter(3 + q, NCHUNK - 6 + q)
        for q in range(3):
            wait_scatter(6 + q, NCHUNK - 3 + q)
        plsc.subcore_barrier()

        @pl.when(s < NS - 1)
        def _pub():
            pltpu.sync_copy(acc.at[pl.ds(s * RPT, RPT)],
                            dst.at[pl.ds(b_off + s * RPT, RPT)])

        @pl.when(s == NS - 1)
        def _pub_last():
            pltpu.sync_copy(acc.at[pl.ds(s * RPT, RPT_LAST)],
                            dst.at[pl.ds(b_off + s * RPT, RPT_LAST)])

        plsc.subcore_barrier()

    for src, dst in ((xlo_hbm, s1lo_hbm), (xhi_hbm, s1hi_hbm)):
        for i_b in range(2):
            run_pass(src, dst, 2 * c + i_b)
    for src, dst in ((s1lo_hbm, s2lo_hbm), (s1hi_hbm, s2hi_hbm)):
        for i_b in range(2):
            run_pass(src, dst, 2 * c + i_b)


_sc_spmm = functools.partial(
    pl.kernel,
    out_type=[jax.ShapeDtypeStruct((B * V, FH), jnp.float32)] * 4,
    mesh=plsc.VectorSubcoreMesh(core_axis_name="c", subcore_axis_name="s"),
    compiler_params=pltpu.CompilerParams(use_tc_tiling_on_sc=False),
    scratch_types=[
        pltpu.VMEM_SHARED((ACCV, FH), jnp.float32),
        pltpu.VMEM((NCHUNK, G), jnp.int32),
        pltpu.VMEM((NCHUNK, G), jnp.int32),
        pltpu.VMEM((9, G), jnp.int32),
        pltpu.VMEM((9, G, FH), jnp.float32),
        pltpu.VMEM((9, G, 16), jnp.float32),
    ] + [pltpu.SemaphoreType.DMA] * 18,
)(_sc_spmm_body)


def _mm_body(x0_ref, s1lo_ref, s1hi_ref, s2lo_ref, s2hi_ref, w_ref, b_ref,
             o_ref):
    acc = jnp.dot(x0_ref[...], w_ref[0], preferred_element_type=jnp.float32)
    acc += jnp.dot(s1lo_ref[...], w_ref[1, :FH], preferred_element_type=jnp.float32)
    acc += jnp.dot(s1hi_ref[...], w_ref[1, FH:], preferred_element_type=jnp.float32)
    acc += jnp.dot(s2lo_ref[...], w_ref[2, :FH], preferred_element_type=jnp.float32)
    acc += jnp.dot(s2hi_ref[...], w_ref[2, FH:], preferred_element_type=jnp.float32)
    o_ref[...] = acc + b_ref[...]


_MM_BLK = 800


def _tc_matmul(x0, s1lo, s1hi, s2lo, s2hi, wadj, bias2d):
    grid = (B * V) // _MM_BLK
    half_spec = pl.BlockSpec((_MM_BLK, FH), lambda i: (i, 0))
    return pl.pallas_call(
        _mm_body,
        grid=(grid,),
        in_specs=[
            pl.BlockSpec((_MM_BLK, F), lambda i: (i, 0)),
            half_spec, half_spec, half_spec, half_spec,
            pl.BlockSpec((3, F, F), lambda i: (0, 0, 0)),
            pl.BlockSpec((1, F), lambda i: (0, 0)),
        ],
        out_specs=pl.BlockSpec((_MM_BLK, F), lambda i: (i, 0)),
        out_shape=jax.ShapeDtypeStruct((B * V, F), jnp.float32),
    )(x0, s1lo, s1hi, s2lo, s2hi, wadj, bias2d)


def _pad_edges(arr, fill):
    slab = arr.reshape(NS, EPT)
    slab = jnp.pad(slab, ((0, 0), (0, EPT_PAD - EPT)), constant_values=fill)
    return slab.reshape(NS, NCHUNK, G)


def kernel(inputs, lap_rows, lap_cols, lap_vals, weight, bias):
    x_flat = inputs.reshape(B * V, F)
    x_lo = x_flat[:, :FH]
    x_hi = x_flat[:, FH:]
    cols_p = _pad_edges(lap_cols.astype(jnp.int32), 0)
    rows_p = _pad_edges(lap_rows.astype(jnp.int32), 0)
    vals_p = _pad_edges(lap_vals, 0.0)
    vals_bc = jnp.broadcast_to(vals_p.reshape(NS * NCHUNK * G, 1),
                               (NS * NCHUNK * G, 16))
    zeros = jnp.zeros((RPT, FH), jnp.float32)
    s1lo, s1hi, s2lo, s2hi = _sc_spmm(x_lo, x_hi, cols_p, rows_p, vals_bc,
                                      zeros)
    wadj = jnp.stack([weight[:, 0, :] - weight[:, 2, :],
                      weight[:, 1, :],
                      2.0 * weight[:, 2, :]], axis=0)
    out_flat = _tc_matmul(x_flat, s1lo, s1hi, s2lo, s2hi, wadj,
                          bias.reshape(1, F))
    return out_flat.reshape(B, V, F)
